# Initial kernel scaffold; baseline (speedup 1.0000x reference)
import jax
import jax.numpy as jnp
from jax.experimental import pallas as pl


def kernel(x, edge_index, p, W0, W_ih, W_hh, b_ih, b_hh, conv_w, conv_b):
    raise NotImplementedError("write your pallas kernel here")


# speedup vs baseline: 19.3387x; 19.3387x over previous
import functools

import jax
import jax.numpy as jnp
from jax import lax
from jax.experimental import pallas as pl
from jax.experimental.pallas import tpu as pltpu
from jax.experimental.pallas import tpu_sc as plsc

N = 10000
C = 128
E = 320000

NUM_SC = 2
NUM_TILES = 16
LANES = 16
NW = NUM_SC * NUM_TILES
EPW = E // NW
CH = 128
NBUF = 2
NFULL = EPW // CH
NGROUP = NFULL // NBUF
TAIL = EPW - NFULL * CH
NP = 10240
RPT = NP // NUM_TILES

BM = 1000
GRID = N // BM


def _z_body(x_ref, p_ref, o_ref):
    p = p_ref[...]
    rows2 = lax.broadcasted_iota(jnp.int32, (C, C), 0)
    cols2 = lax.broadcasted_iota(jnp.int32, (C, C), 1)
    diag_p = jnp.where(rows2 == cols2, jnp.broadcast_to(p, (C, C)), 0.0)
    p_alt = jnp.dot(diag_p, jnp.ones((C, C), jnp.float32),
                    preferred_element_type=jnp.float32)
    x32r = x_ref[...].astype(jnp.bfloat16).astype(jnp.float32)
    pr32 = p_alt.astype(jnp.bfloat16).astype(jnp.float32)
    z_mat = jnp.dot(x32r, pr32, preferred_element_type=jnp.float32)
    o_ref[...] = jnp.max(z_mat, axis=1).reshape(80, 125)


def _tc_z(x, p2, interpret=False):
    return pl.pallas_call(
        _z_body,
        out_shape=jax.ShapeDtypeStruct((80, 125), jnp.float32),
        interpret=interpret,
    )(x, p2)


def _prep_body(score_in_ref, x_ref, w0_ref, wih_ref, whh_ref, bih_ref,
               bhh_ref, w_ref, score_ref, xt_ref, vals_ref, perm_ref):
    score_ref[...] = score_in_ref[...]

    rows = lax.broadcasted_iota(jnp.int32, (80, 125), 0)
    cols = lax.broadcasted_iota(jnp.int32, (80, 125), 1)
    lin = rows * 125 + cols

    def tk_body(k, _):
        a = score_ref[...]
        m = jnp.max(a)
        pos = jnp.min(jnp.where(a == m, lin, jnp.int32(2 ** 30)))
        vals_ref[k] = m
        perm_ref[k] = pos
        score_ref[...] = jnp.where(lin == pos, -2.0, a)
        return 0

    lax.fori_loop(0, C, tk_body, 0)

    def gather_body(k, _):
        i = perm_ref[k]
        xt_ref[pl.ds(k, 1), :] = x_ref[pl.ds(i, 1), :] * vals_ref[k]
        return 0

    lax.fori_loop(0, C, gather_body, 0)

    xt = xt_ref[...]
    w0 = w0_ref[...]
    dn = (((1,), (1,)), ((), ()))
    gi = lax.dot_general(xt.astype(jnp.bfloat16),
                         wih_ref[...].astype(jnp.bfloat16), dn,
                         preferred_element_type=jnp.float32) + bih_ref[...]
    gh = lax.dot_general(w0.astype(jnp.bfloat16),
                         whh_ref[...].astype(jnp.bfloat16), dn,
                         preferred_element_type=jnp.float32) + bhh_ref[...]
    r = jax.nn.sigmoid(gi[:, :C] + gh[:, :C])
    z = jax.nn.sigmoid(gi[:, C:2 * C] + gh[:, C:2 * C])
    n_g = jnp.tanh(gi[:, 2 * C:] + r * gh[:, 2 * C:])
    w_ref[...] = (1.0 - z) * n_g + z * w0


def _tc_prep(score2d, x, w0, wih, whh, bih2, bhh2, interpret=False):
    return pl.pallas_call(
        _prep_body,
        out_shape=jax.ShapeDtypeStruct((C, C), jnp.float32),
        scratch_shapes=[
            pltpu.VMEM((80, 125), jnp.float32),
            pltpu.VMEM((C, C), jnp.float32),
            pltpu.SMEM((C,), jnp.float32),
            pltpu.SMEM((C,), jnp.int32),
        ],
        interpret=interpret,
    )(score2d, x, w0, wih, whh, bih2, bhh2)


_BMD = 1024


def _dis_body(cnt_ref, o_ref):
    o_ref[...] = lax.rsqrt(cnt_ref[0] + cnt_ref[1] + 1.0)


def _tc_dis(cnt, interpret=False):
    return pl.pallas_call(
        _dis_body,
        out_shape=jax.ShapeDtypeStruct((NP, C), jnp.float32),
        grid=(NP // _BMD,),
        in_specs=[pl.BlockSpec((NUM_SC, _BMD, C), lambda i: (0, i, 0))],
        out_specs=pl.BlockSpec((_BMD, C), lambda i: (i, 0)),
        interpret=interpret,
    )(cnt)


def _mm_scale_body(x_ref, w_ref, dis_ref, o_ref):
    o_ref[...] = jnp.dot(x_ref[...].astype(jnp.bfloat16),
                         w_ref[...].astype(jnp.bfloat16),
                         preferred_element_type=jnp.float32) * dis_ref[...]


def _tc_mm_scale(x, w, dis, interpret=False):
    return pl.pallas_call(
        _mm_scale_body,
        out_shape=jax.ShapeDtypeStruct((N, C), jnp.float32),
        grid=(GRID,),
        in_specs=[
            pl.BlockSpec((BM, C), lambda i: (i, 0)),
            pl.BlockSpec((C, C), lambda i: (0, 0)),
            pl.BlockSpec((BM, C), lambda i: (i, 0)),
        ],
        out_specs=pl.BlockSpec((BM, C), lambda i: (i, 0)),
        interpret=interpret,
    )(x, w, dis)


def _mid_body(acc_ref, y_ref, dis_ref, cw_ref, o_ref):
    dis = dis_ref[...]
    h = dis * (acc_ref[0] + acc_ref[1] + y_ref[...])
    o_ref[...] = jnp.dot(jnp.maximum(h, 0.0).astype(jnp.bfloat16),
                         cw_ref[...].astype(jnp.bfloat16),
                         preferred_element_type=jnp.float32) * dis


def _tc_mid(acc, y, dis, conv_w, interpret=False):
    return pl.pallas_call(
        _mid_body,
        out_shape=jax.ShapeDtypeStruct((N, C), jnp.float32),
        grid=(GRID,),
        in_specs=[
            pl.BlockSpec((NUM_SC, BM, C), lambda i: (0, i, 0)),
            pl.BlockSpec((BM, C), lambda i: (i, 0)),
            pl.BlockSpec((BM, C), lambda i: (i, 0)),
            pl.BlockSpec((C, C), lambda i: (0, 0)),
        ],
        out_specs=pl.BlockSpec((BM, C), lambda i: (i, 0)),
        interpret=interpret,
    )(acc, y, dis, conv_w)


def _fin_body(acc_ref, y_ref, dis_ref, cb_ref, o_ref):
    o_ref[...] = dis_ref[...] * (acc_ref[0] + acc_ref[1] + y_ref[...]) \
        + cb_ref[...]


def _tc_fin(acc, y, dis, cb2, interpret=False):
    return pl.pallas_call(
        _fin_body,
        out_shape=jax.ShapeDtypeStruct((N, C), jnp.float32),
        grid=(GRID,),
        in_specs=[
            pl.BlockSpec((NUM_SC, BM, C), lambda i: (0, i, 0)),
            pl.BlockSpec((BM, C), lambda i: (i, 0)),
            pl.BlockSpec((BM, C), lambda i: (i, 0)),
            pl.BlockSpec((1, C), lambda i: (0, 0)),
        ],
        out_specs=pl.BlockSpec((BM, C), lambda i: (i, 0)),
        interpret=interpret,
    )(acc, y, dis, cb2)


def _sc_mesh():
    return plsc.VectorSubcoreMesh(core_axis_name="c", subcore_axis_name="s",
                                  num_cores=NUM_SC, num_subcores=NUM_TILES)


def _histc_body(dst_hbm, cnt_out, didx, tidx, ones_v, acc_sh, sem):
    c = lax.axis_index("c")
    s = lax.axis_index("s")
    wid = c * NUM_TILES + s

    zeros16 = jnp.zeros((LANES,), jnp.float32)
    ones16 = jnp.ones((LANES,), jnp.float32)

    def zfill_body(i, _):
        for j in range(C // LANES):
            ones_v[i, pl.ds(j * LANES, LANES)] = zeros16
        return 0

    lax.fori_loop(0, CH, zfill_body, 0)

    base = s * RPT
    for k in range(RPT // CH):
        pltpu.sync_copy(ones_v, acc_sh.at[pl.ds(base + k * CH, CH)])
    plsc.subcore_barrier()

    def ofill_body(i, _):
        for j in range(C // LANES):
            ones_v[i, pl.ds(j * LANES, LANES)] = ones16
        return 0

    lax.fori_loop(0, CH, ofill_body, 0)

    ebase = wid * EPW

    def chunk_body(t, _):
        pltpu.sync_copy(dst_hbm.at[pl.ds(ebase + t * CH, CH)], didx.at[0])
        pltpu.sync_copy(ones_v, acc_sh.at[didx.at[0]], add=True)
        return 0

    lax.fori_loop(0, NFULL, chunk_body, 0)

    pltpu.sync_copy(dst_hbm.at[pl.ds(ebase + NFULL * CH, TAIL)], tidx.at[0])
    pltpu.sync_copy(ones_v.at[pl.ds(0, TAIL)], acc_sh.at[tidx.at[0]],
                    add=True)

    plsc.subcore_barrier()
    pltpu.sync_copy(acc_sh.at[pl.ds(base, RPT)],
                    cnt_out.at[c].at[pl.ds(base, RPT)])


def _sc_histc(dst, interpret=False):
    return pl.kernel(
        _histc_body,
        out_type=jax.ShapeDtypeStruct((NUM_SC, NP, C), jnp.float32),
        mesh=_sc_mesh(),
        scratch_types=[
            pltpu.VMEM((1, CH), jnp.int32),
            pltpu.VMEM((1, TAIL), jnp.int32),
            pltpu.VMEM((CH, C), jnp.float32),
            pltpu.VMEM_SHARED((NP, C), jnp.float32),
            pltpu.SemaphoreType.DMA,
        ],
        interpret=interpret,
    )(dst)


def _scatter_kernel_body(y_hbm, src_hbm, dst_hbm, acc_out,
                         sidx, didx, tsidx, tdidx, rows, acc_sh, gsem):
    c = lax.axis_index("c")
    s = lax.axis_index("s")
    wid = c * NUM_TILES + s

    zeros16 = jnp.zeros((LANES,), jnp.float32)

    def zfill_body(i, _):
        for j in range(C // LANES):
            rows[0, i, pl.ds(j * LANES, LANES)] = zeros16
        return 0

    lax.fori_loop(0, CH, zfill_body, 0)

    base = s * RPT
    for k in range(RPT // CH):
        pltpu.sync_copy(rows.at[0], acc_sh.at[pl.ds(base + k * CH, CH)])
    plsc.subcore_barrier()

    ebase = wid * EPW

    def group_body(g, _):
        t0 = g * NBUF
        descs = []
        for b in range(NBUF):
            off = ebase + (t0 + b) * CH
            pltpu.sync_copy(src_hbm.at[pl.ds(off, CH)], sidx.at[b])
            pltpu.sync_copy(dst_hbm.at[pl.ds(off, CH)], didx.at[b])
            descs.append(pltpu.async_copy(y_hbm.at[sidx.at[b]], rows.at[b],
                                          gsem))
        for b in range(NBUF):
            descs[b].wait()
            pltpu.sync_copy(rows.at[b], acc_sh.at[didx.at[b]], add=True)
        return 0

    lax.fori_loop(0, NGROUP, group_body, 0)

    toff = ebase + NFULL * CH
    pltpu.sync_copy(src_hbm.at[pl.ds(toff, TAIL)], tsidx.at[0])
    pltpu.sync_copy(dst_hbm.at[pl.ds(toff, TAIL)], tdidx.at[0])
    pltpu.async_copy(y_hbm.at[tsidx.at[0]], rows.at[0].at[pl.ds(0, TAIL)],
                     gsem).wait()
    pltpu.sync_copy(rows.at[0].at[pl.ds(0, TAIL)], acc_sh.at[tdidx.at[0]],
                    add=True)

    plsc.subcore_barrier()
    pltpu.sync_copy(acc_sh.at[pl.ds(base, RPT)],
                    acc_out.at[c].at[pl.ds(base, RPT)])


def _sc_scatter(y, src, dst, interpret=False):
    return pl.kernel(
        _scatter_kernel_body,
        out_type=jax.ShapeDtypeStruct((NUM_SC, NP, C), jnp.float32),
        mesh=_sc_mesh(),
        scratch_types=[
            pltpu.VMEM((NBUF, CH), jnp.int32),
            pltpu.VMEM((NBUF, CH), jnp.int32),
            pltpu.VMEM((1, TAIL), jnp.int32),
            pltpu.VMEM((1, TAIL), jnp.int32),
            pltpu.VMEM((NBUF, CH, C), jnp.float32),
            pltpu.VMEM_SHARED((NP, C), jnp.float32),
            pltpu.SemaphoreType.DMA,
        ],
        interpret=interpret,
    )(y, src, dst)


def kernel(x, edge_index, p, W0, W_ih, W_hh, b_ih, b_hh, conv_w, conv_b):
    src = edge_index[0].astype(jnp.int32)
    dst = edge_index[1].astype(jnp.int32)
    p2 = p.reshape(1, C)
    bih2 = b_ih.reshape(1, 3 * C)
    bhh2 = b_hh.reshape(1, 3 * C)
    cb2 = conv_b.reshape(1, C)

    cnt = _sc_histc(dst)
    dis = _tc_dis(cnt)
    z2d = _tc_z(x, p2)
    score2d = jnp.tanh(z2d / jnp.linalg.norm(p))
    w_evolved = _tc_prep(score2d, x, W0, W_ih, W_hh, bih2, bhh2)
    y = _tc_mm_scale(x, w_evolved, dis)
    acc = _sc_scatter(y, src, dst)
    y2 = _tc_mid(acc, y, dis, conv_w)
    acc2 = _sc_scatter(y2, src, dst)
    out = _tc_fin(acc2, y2, dis, cb2)
    return out
